# baseline (speedup 1.0000x reference)
import functools

import jax
import jax.numpy as jnp
from jax import lax
from jax.experimental import pallas as pl
from jax.experimental.pallas import tpu as pltpu
from jax.experimental.pallas import tpu_sc as plsc

K = 16
KP1 = K + 1


def _transpose_kernel(feat_ref, out_ref):
    out_ref[0] = feat_ref[0].T


def _topk_kernel(coords_ref, idx_ref, *, rb: int, n: int, boff: int = 0):
    b = pl.program_id(0) + boff
    i = pl.program_id(1)
    c = coords_ref[0]
    rows = coords_ref[0, :, pl.ds(i * rb, rb)]
    col_sq = jnp.sum(c * c, axis=0, keepdims=True)
    row_sq = jnp.sum(rows * rows, axis=0)
    mm = lax.dot_general(
        rows, c, (((0,), (0,)), ((), ())),
        preferred_element_type=jnp.float32,
    )
    dist = -2.0 * mm
    dist = dist + row_sq[:, None]
    dist = dist + col_sq
    dist = jnp.clip(dist, 1e-12, None)

    iota_n = lax.broadcasted_iota(jnp.int32, (rb, n), 1)
    vals = dist
    for k in range(KP1):
        m = jnp.min(vals, axis=1, keepdims=True)
        am = jnp.min(jnp.where(vals == m, iota_n, n), axis=1, keepdims=True)
        idx_ref[0, k, :] = am[:, 0] + b * n
        if k + 1 < KP1:
            vals = jnp.where(iota_n == am, jnp.inf, vals)


def _make_sc_gather(rows: int, c: int, nw: int, ch: int):
    per_w = rows // nw
    mesh = plsc.VectorSubcoreMesh(core_axis_name="c", subcore_axis_name="s")

    @functools.partial(
        pl.kernel, mesh=mesh,
        out_type=jax.ShapeDtypeStruct((rows, c), jnp.float32),
        scratch_types=[
            pltpu.VMEM((ch,), jnp.int32),
            pltpu.VMEM((ch, c), jnp.float32),
            pltpu.SemaphoreType.DMA,
        ],
    )
    def sc_gather(table_hbm, ids_hbm, out_hbm, idx_v, rows_v, sem):
        wid = lax.axis_index("s") * 2 + lax.axis_index("c")
        base = wid * per_w

        def body(t, carry):
            off = base + t * ch
            pltpu.sync_copy(ids_hbm.at[pl.ds(off, ch)], idx_v)
            pltpu.async_copy(table_hbm.at[idx_v], rows_v, sem).wait()
            pltpu.sync_copy(rows_v, out_hbm.at[pl.ds(off, ch)])
            return carry

        lax.fori_loop(0, per_w // ch, body, 0)

    return sc_gather


def _finalize_kernel(gath_ref, feat_ref, out_ref, *, nb: int, c: int):
    j = pl.program_id(1)
    fb = feat_ref[0, :, pl.ds(j * nb, nb)]
    for k in range(K):
        slab = gath_ref[0, k]
        g = slab.T
        out_ref[0, 0:c, k, :] = fb
        out_ref[0, c:2 * c, k, :] = g - fb


def _finalize_alias_kernel(gath_ref, feat_ref, prev_ref, out_ref, *, nb, c):
    del prev_ref
    _finalize_kernel(gath_ref, feat_ref, out_ref, nb=nb, c=c)


def kernel(coordinates, features):
    if features.ndim == 4 and features.shape[-1] == 1:
        features = jnp.squeeze(features, axis=-1)
    B, C, N = features.shape
    RB = 256
    NB = 512
    ROWS = B * K * N

    table = pl.pallas_call(
        _transpose_kernel,
        grid=(B,),
        in_specs=[pl.BlockSpec((1, C, N), lambda b: (b, 0, 0))],
        out_specs=pl.BlockSpec((1, N, C), lambda b: (b, 0, 0)),
        out_shape=jax.ShapeDtypeStruct((B, N, C), jnp.float32),
    )(features)

    S = 8
    H = B // S
    HROWS = H * K * N
    table_flat = table.reshape(B * N, C)
    sc_gather = _make_sc_gather(HROWS, C, 32, 128)

    def topk_split(boff):
        return pl.pallas_call(
            functools.partial(_topk_kernel, rb=RB, n=N, boff=boff),
            grid=(H, N // RB),
            in_specs=[pl.BlockSpec((1, 3, N), lambda b, i: (b, 0, 0))],
            out_specs=pl.BlockSpec((1, KP1, RB), lambda b, i: (b, 0, i)),
            out_shape=jax.ShapeDtypeStruct((H, KP1, N), jnp.int32),
            compiler_params=pltpu.CompilerParams(
                dimension_semantics=("parallel", "parallel")),
        )(lax.slice_in_dim(coordinates, boff, boff + H, axis=0))

    idxs = [topk_split(s * H) for s in range(S)]
    gaths = [sc_gather(table_flat, ix[:, 1:, :].reshape(HROWS))
             for ix in idxs]

    out_shape = jax.ShapeDtypeStruct((B, 2 * C, K, N), jnp.float32)
    out = None
    for s in range(S):
        boff = s * H
        gspec = pl.BlockSpec((1, K, NB, C), lambda b, j: (b, 0, j, 0))
        fspec = pl.BlockSpec((1, C, N), lambda b, j, o=boff: (b + o, 0, 0))
        ospec = pl.BlockSpec(
            (1, 2 * C, K, NB), lambda b, j, o=boff: (b + o, 0, 0, j))
        gath = gaths[s].reshape(H, K, N, C)
        if s == 0:
            out = pl.pallas_call(
                functools.partial(_finalize_kernel, nb=NB, c=C),
                grid=(H, N // NB),
                in_specs=[gspec, fspec],
                out_specs=ospec,
                out_shape=out_shape,
                compiler_params=pltpu.CompilerParams(
                    dimension_semantics=("parallel", "parallel")),
            )(gath, features)
        else:
            out = pl.pallas_call(
                functools.partial(_finalize_alias_kernel, nb=NB, c=C),
                grid=(H, N // NB),
                in_specs=[gspec, fspec, pl.BlockSpec(memory_space=pl.ANY)],
                out_specs=ospec,
                out_shape=out_shape,
                input_output_aliases={2: 0},
                compiler_params=pltpu.CompilerParams(
                    dimension_semantics=("parallel", "parallel")),
            )(gath, features, out)
    return out.transpose(0, 1, 3, 2)


# speedup vs baseline: 1.0173x; 1.0173x over previous
import functools

import jax
import jax.numpy as jnp
from jax import lax
from jax.experimental import pallas as pl
from jax.experimental.pallas import tpu as pltpu
from jax.experimental.pallas import tpu_sc as plsc

K = 16
KP1 = K + 1


def _transpose_kernel(feat_ref, out_ref):
    out_ref[0] = feat_ref[0].T


def _topk_kernel(coords_ref, idx_ref, *, rb: int, n: int, boff: int = 0):
    b = pl.program_id(0) + boff
    i = pl.program_id(1)
    c = coords_ref[0]
    rows = coords_ref[0, :, pl.ds(i * rb, rb)]
    col_sq = jnp.sum(c * c, axis=0, keepdims=True)
    row_sq = jnp.sum(rows * rows, axis=0)
    mm = lax.dot_general(
        rows, c, (((0,), (0,)), ((), ())),
        preferred_element_type=jnp.float32,
    )
    dist = -2.0 * mm
    dist = dist + row_sq[:, None]
    dist = dist + col_sq
    dist = jnp.clip(dist, 1e-12, None)

    iota_n = lax.broadcasted_iota(jnp.int32, (rb, n), 1)
    vals = dist
    for k in range(KP1):
        m = jnp.min(vals, axis=1, keepdims=True)
        am = jnp.min(jnp.where(vals == m, iota_n, n), axis=1, keepdims=True)
        idx_ref[0, k, :] = am[:, 0] + b * n
        if k + 1 < KP1:
            vals = jnp.where(iota_n == am, jnp.inf, vals)


def _make_sc_gather(rows: int, c: int, nw: int, ch: int):
    per_w = rows // nw
    mesh = plsc.VectorSubcoreMesh(core_axis_name="c", subcore_axis_name="s")

    @functools.partial(
        pl.kernel, mesh=mesh,
        out_type=jax.ShapeDtypeStruct((rows, c), jnp.float32),
        scratch_types=[
            pltpu.VMEM((ch,), jnp.int32),
            pltpu.VMEM((ch, c), jnp.float32),
            pltpu.SemaphoreType.DMA,
        ],
    )
    def sc_gather(table_hbm, ids_hbm, out_hbm, idx_v, rows_v, sem):
        wid = lax.axis_index("s") * 2 + lax.axis_index("c")
        base = wid * per_w

        def body(t, carry):
            off = base + t * ch
            pltpu.sync_copy(ids_hbm.at[pl.ds(off, ch)], idx_v)
            pltpu.async_copy(table_hbm.at[idx_v], rows_v, sem).wait()
            pltpu.sync_copy(rows_v, out_hbm.at[pl.ds(off, ch)])
            return carry

        lax.fori_loop(0, per_w // ch, body, 0)

    return sc_gather


def _finalize_kernel(gath_ref, feat_ref, out_ref, *, nb: int, c: int):
    j = pl.program_id(1)
    fb = feat_ref[0, :, pl.ds(j * nb, nb)]
    for k in range(K):
        slab = gath_ref[0, k]
        g = slab.T
        out_ref[0, 0:c, k, :] = fb
        out_ref[0, c:2 * c, k, :] = g - fb


def _finalize_alias_kernel(gath_ref, feat_ref, prev_ref, out_ref, *, nb, c):
    del prev_ref
    _finalize_kernel(gath_ref, feat_ref, out_ref, nb=nb, c=c)


def kernel(coordinates, features):
    if features.ndim == 4 and features.shape[-1] == 1:
        features = jnp.squeeze(features, axis=-1)
    B, C, N = features.shape
    RB = 256
    NB = 512
    ROWS = B * K * N

    table = pl.pallas_call(
        _transpose_kernel,
        grid=(B,),
        in_specs=[pl.BlockSpec((1, C, N), lambda b: (b, 0, 0))],
        out_specs=pl.BlockSpec((1, N, C), lambda b: (b, 0, 0)),
        out_shape=jax.ShapeDtypeStruct((B, N, C), jnp.float32),
    )(features)

    S = 4
    H = B // S
    HROWS = H * K * N
    table_flat = table.reshape(B * N, C)
    sc_gather = _make_sc_gather(HROWS, C, 32, 128)

    def topk_split(boff):
        return pl.pallas_call(
            functools.partial(_topk_kernel, rb=RB, n=N, boff=boff),
            grid=(H, N // RB),
            in_specs=[pl.BlockSpec((1, 3, N), lambda b, i: (b, 0, 0))],
            out_specs=pl.BlockSpec((1, KP1, RB), lambda b, i: (b, 0, i)),
            out_shape=jax.ShapeDtypeStruct((H, KP1, N), jnp.int32),
            compiler_params=pltpu.CompilerParams(
                dimension_semantics=("parallel", "parallel")),
        )(lax.slice_in_dim(coordinates, boff, boff + H, axis=0))

    idxs = [topk_split(s * H) for s in range(S)]
    gaths = [sc_gather(table_flat, ix[:, 1:, :].reshape(HROWS))
             for ix in idxs]

    out_shape = jax.ShapeDtypeStruct((B, 2 * C, K, N), jnp.float32)
    out = None
    for s in range(S):
        boff = s * H
        gspec = pl.BlockSpec((1, K, NB, C), lambda b, j: (b, 0, j, 0))
        fspec = pl.BlockSpec((1, C, N), lambda b, j, o=boff: (b + o, 0, 0))
        ospec = pl.BlockSpec(
            (1, 2 * C, K, NB), lambda b, j, o=boff: (b + o, 0, 0, j))
        gath = gaths[s].reshape(H, K, N, C)
        if s == 0:
            out = pl.pallas_call(
                functools.partial(_finalize_kernel, nb=NB, c=C),
                grid=(H, N // NB),
                in_specs=[gspec, fspec],
                out_specs=ospec,
                out_shape=out_shape,
                compiler_params=pltpu.CompilerParams(
                    dimension_semantics=("parallel", "parallel")),
            )(gath, features)
        else:
            out = pl.pallas_call(
                functools.partial(_finalize_alias_kernel, nb=NB, c=C),
                grid=(H, N // NB),
                in_specs=[gspec, fspec, pl.BlockSpec(memory_space=pl.ANY)],
                out_specs=ospec,
                out_shape=out_shape,
                input_output_aliases={2: 0},
                compiler_params=pltpu.CompilerParams(
                    dimension_semantics=("parallel", "parallel")),
            )(gath, features, out)
    return out.transpose(0, 1, 3, 2)
